# baseline (speedup 1.0000x reference)
import functools

import jax
import jax.numpy as jnp
from jax import lax
from jax.experimental import pallas as pl
from jax.experimental.pallas import tpu as pltpu
from jax.experimental.pallas import tpu_sc as plsc

L = 48
W = 7
B = 4
D = 4 * (L - 1) + 1
WVOL = W * W * W

NCORES = 1
NSUB = 16
NW = NCORES * NSUB
WIN_PER_W = (B * L) // NW
SUB_PER_B = L // WIN_PER_W

ROWS_PER_B = 4
MASK_OFF = 64


def _sc_windows(packed):
    mesh = plsc.VectorSubcoreMesh(
        core_axis_name="c", subcore_axis_name="s",
        num_cores=NCORES, num_subcores=NSUB,
    )

    @functools.partial(
        pl.kernel,
        mesh=mesh,
        out_type=jax.ShapeDtypeStruct((B, L, W, W, W, 1), jnp.float32),
        compiler_params=pltpu.CompilerParams(
            needs_layout_passes=False, use_tc_tiling_on_sc=False,
            disable_bounds_checks=True, disable_semaphore_checks=True),
        scratch_types=[
            pltpu.VMEM((ROWS_PER_B, 128), jnp.int32),
            pltpu.VMEM((WIN_PER_W, W, W, W, 1), jnp.float32),
            pltpu.SemaphoreType.DMA,
        ],
    )
    def body(pk_hbm, out_hbm, pk_v, win_v, sem):
        wid = lax.axis_index("s") * NCORES + lax.axis_index("c")
        b = wid // SUB_PER_B
        i0 = (wid % SUB_PER_B) * WIN_PER_W

        cp = pltpu.async_copy(
            pk_hbm.at[pl.ds(b * ROWS_PER_B, ROWS_PER_B)], pk_v, sem)

        lane = jnp.arange(16, dtype=jnp.int32)
        zeros = jnp.zeros((16,), jnp.float32)
        zero_i = jnp.zeros((16,), jnp.int32)
        for w in range(WIN_PER_W):
            roww = jnp.full((16,), w, jnp.int32)
            for k in range((WVOL + 15) // 16):
                p = jnp.minimum(k * 16 + lane, WVOL - 1)
                px = p // (W * W)
                pr = p - px * (W * W)
                plsc.store_scatter(
                    win_v, [roww, px, pr // W, pr - (pr // W) * W, zero_i],
                    zeros)

        cp.wait()

        def gat(q):
            return plsc.load_gather(pk_v, [(q >> 7) + 1, q & 127])

        def acid(col):
            return plsc.bitcast(pk_v[0, pl.ds(col, 16)], jnp.float32)

        def msk(col):
            return plsc.bitcast(pk_v[0, pl.ds(MASK_OFF + col, 16)], jnp.float32)

        chunks = []
        for t in range(3):
            pid3 = (t * 16) * 3 + lane * 3
            cx = 2 * (gat(pid3) + (L - 1))
            cy = 2 * (gat(pid3 + 1) + (L - 1))
            cz = 2 * (gat(pid3 + 2) + (L - 1))
            v = acid(t * 16) * msk(t * 16)
            valid = lane < 16
            chunks.append((cx, cy, cz, v, valid))
        for t in range(3):
            mid = t * 16 + lane
            valid = mid < (L - 1)
            pid3 = jnp.where(valid, mid, L - 2) * 3
            cx = gat(pid3) + gat(pid3 + 3) + 2 * (L - 1)
            cy = gat(pid3 + 1) + gat(pid3 + 4) + 2 * (L - 1)
            cz = gat(pid3 + 2) + gat(pid3 + 5) + 2 * (L - 1)
            v = (acid(t * 16) + acid(t * 16 + 1) + 1.0) * msk(t * 16 + 1)
            chunks.append((cx, cy, cz, v, valid))

        for w in range(WIN_PER_W):
            i3 = (i0 + w) * 3
            sx = jnp.clip(
                2 * (gat(jnp.full((16,), i3, jnp.int32)) + (L - 1))
                - W // 2, 0, D - W)
            sy = jnp.clip(
                2 * (gat(jnp.full((16,), i3 + 1, jnp.int32)) + (L - 1))
                - W // 2, 0, D - W)
            sz = jnp.clip(
                2 * (gat(jnp.full((16,), i3 + 2, jnp.int32)) + (L - 1))
                - W // 2, 0, D - W)
            row = jnp.full((16,), w, jnp.int32)
            for (cx, cy, cz, v, valid) in chunks:
                dx = cx - sx
                dy = cy - sy
                dz = cz - sz
                m = (
                    (dx >= 0) & (dx < W)
                    & (dy >= 0) & (dy < W)
                    & (dz >= 0) & (dz < W)
                    & valid
                )
                dx = jnp.where(m, dx, 0)
                dy = jnp.where(m, dy, 0)
                dz = jnp.where(m, dz, 0)
                plsc.addupdate_scatter(
                    win_v, [row, dx, dy, dz, zero_i], v, mask=m)

        pltpu.sync_copy(win_v, out_hbm.at[b, pl.ds(i0, WIN_PER_W)])

    return body(packed)


def kernel(acids, mask, idx):
    acids_i = lax.bitcast_convert_type(acids.astype(jnp.float32), jnp.int32)
    mask_i = lax.bitcast_convert_type(mask.astype(jnp.float32), jnp.int32)
    idx_i = idx.astype(jnp.int32).reshape(B, 3 * L)

    z16 = jnp.zeros((B, 16), jnp.int32)
    row0 = jnp.concatenate([acids_i, z16, mask_i, z16], axis=1)
    rows12 = jnp.concatenate(
        [idx_i, jnp.zeros((B, 256 - 3 * L), jnp.int32)], axis=1
    ).reshape(B, 2, 128)
    row3 = jnp.zeros((B, 1, 128), jnp.int32)
    packed = jnp.concatenate(
        [row0[:, None, :], rows12, row3], axis=1
    ).reshape(ROWS_PER_B * B, 128)

    return _sc_windows(packed)


# speedup vs baseline: 2.8872x; 2.8872x over previous
import functools

import jax
import jax.numpy as jnp
from jax import lax
from jax.experimental import pallas as pl
from jax.experimental.pallas import tpu as pltpu
from jax.experimental.pallas import tpu_sc as plsc

L = 48
W = 7
B = 4
D = 4 * (L - 1) + 1
WVOL = W * W * W

NCORES = 1
NSUB = 16
NW = NCORES * NSUB
WIN_PER_W = (B * L) // NW
SUB_PER_B = L // WIN_PER_W

ROWS_PER_B = 4
MASK_OFF = 64


def _sc_windows(packed):
    mesh = plsc.VectorSubcoreMesh(
        core_axis_name="c", subcore_axis_name="s",
        num_cores=NCORES, num_subcores=NSUB,
    )

    @functools.partial(
        pl.kernel,
        mesh=mesh,
        out_type=jax.ShapeDtypeStruct((B, L, WVOL), jnp.float32),
        compiler_params=pltpu.CompilerParams(
            needs_layout_passes=False, use_tc_tiling_on_sc=False,
            disable_bounds_checks=True, disable_semaphore_checks=True),
        scratch_types=[
            pltpu.VMEM((ROWS_PER_B, 128), jnp.int32),
            pltpu.VMEM((WIN_PER_W, WVOL), jnp.float32),
            pltpu.SemaphoreType.DMA,
        ],
    )
    def body(pk_hbm, out_hbm, pk_v, win_v, sem):
        wid = lax.axis_index("s") * NCORES + lax.axis_index("c")
        b = wid // SUB_PER_B
        i0 = (wid % SUB_PER_B) * WIN_PER_W

        cp = pltpu.async_copy(
            pk_hbm.at[pl.ds(b * ROWS_PER_B, ROWS_PER_B)], pk_v, sem)

        zeros = jnp.zeros((16,), jnp.float32)
        for w in range(WIN_PER_W):
            for k in range(WVOL // 16):
                win_v[w, pl.ds(k * 16, 16)] = zeros
            win_v[w, pl.ds(WVOL - 16, 16)] = zeros

        cp.wait()

        lane = jnp.arange(16, dtype=jnp.int32)

        def gat(q):
            return plsc.load_gather(pk_v, [(q >> 7) + 1, q & 127])

        def acid(col):
            return plsc.bitcast(pk_v[0, pl.ds(col, 16)], jnp.float32)

        def msk(col):
            return plsc.bitcast(pk_v[0, pl.ds(MASK_OFF + col, 16)], jnp.float32)

        chunks = []
        for t in range(3):
            pid3 = (t * 16) * 3 + lane * 3
            cx = 2 * (gat(pid3) + (L - 1))
            cy = 2 * (gat(pid3 + 1) + (L - 1))
            cz = 2 * (gat(pid3 + 2) + (L - 1))
            v = acid(t * 16) * msk(t * 16)
            valid = lane < 16
            chunks.append((cx, cy, cz, v, valid))
        for t in range(3):
            mid = t * 16 + lane
            valid = mid < (L - 1)
            pid3 = jnp.where(valid, mid, L - 2) * 3
            cx = gat(pid3) + gat(pid3 + 3) + 2 * (L - 1)
            cy = gat(pid3 + 1) + gat(pid3 + 4) + 2 * (L - 1)
            cz = gat(pid3 + 2) + gat(pid3 + 5) + 2 * (L - 1)
            v = (acid(t * 16) + acid(t * 16 + 1) + 1.0) * msk(t * 16 + 1)
            chunks.append((cx, cy, cz, v, valid))

        for w in range(WIN_PER_W):
            i3 = (i0 + w) * 3
            sx = jnp.clip(
                2 * (gat(jnp.full((16,), i3, jnp.int32)) + (L - 1))
                - W // 2, 0, D - W)
            sy = jnp.clip(
                2 * (gat(jnp.full((16,), i3 + 1, jnp.int32)) + (L - 1))
                - W // 2, 0, D - W)
            sz = jnp.clip(
                2 * (gat(jnp.full((16,), i3 + 2, jnp.int32)) + (L - 1))
                - W // 2, 0, D - W)
            row = jnp.full((16,), w, jnp.int32)
            for (cx, cy, cz, v, valid) in chunks:
                dx = cx - sx
                dy = cy - sy
                dz = cz - sz
                m = (
                    (dx >= 0) & (dx < W)
                    & (dy >= 0) & (dy < W)
                    & (dz >= 0) & (dz < W)
                    & valid
                )
                off = dx * (W * W) + dy * W + dz
                off = jnp.where(m, off, 0)
                plsc.addupdate_scatter(win_v, [row, off], v, mask=m)

        pltpu.sync_copy(win_v, out_hbm.at[b, pl.ds(i0, WIN_PER_W)])

    return body(packed)


def kernel(acids, mask, idx):
    acids_i = lax.bitcast_convert_type(acids.astype(jnp.float32), jnp.int32)
    mask_i = lax.bitcast_convert_type(mask.astype(jnp.float32), jnp.int32)
    idx_i = idx.astype(jnp.int32).reshape(B, 3 * L)

    z16 = jnp.zeros((B, 16), jnp.int32)
    packed = jnp.concatenate(
        [acids_i, z16, mask_i, z16, idx_i,
         jnp.zeros((B, 512 - 128 - 3 * L), jnp.int32)], axis=1
    ).reshape(ROWS_PER_B * B, 128)

    rows = _sc_windows(packed)
    return rows.reshape(B, L, W, W, W, 1)


# speedup vs baseline: 2.9248x; 1.0130x over previous
import functools

import jax
import jax.numpy as jnp
from jax import lax
from jax.experimental import pallas as pl
from jax.experimental.pallas import tpu as pltpu
from jax.experimental.pallas import tpu_sc as plsc

L = 48
W = 7
B = 4
D = 4 * (L - 1) + 1
WVOL = W * W * W

NCORES = 1
NSUB = 16
NW = NCORES * NSUB
WIN_PER_W = (B * L) // NW
SUB_PER_B = L // WIN_PER_W

ROWS_PER_B = 4
MASK_OFF = 64


def _sc_windows(packed):
    mesh = plsc.VectorSubcoreMesh(
        core_axis_name="c", subcore_axis_name="s",
        num_cores=NCORES, num_subcores=NSUB,
    )

    @functools.partial(
        pl.kernel,
        mesh=mesh,
        out_type=jax.ShapeDtypeStruct((B, L, WVOL), jnp.float32),
        compiler_params=pltpu.CompilerParams(
            needs_layout_passes=False, use_tc_tiling_on_sc=False,
            allow_input_fusion=[True],
            disable_bounds_checks=True, disable_semaphore_checks=True),
        scratch_types=[
            pltpu.VMEM((ROWS_PER_B, 128), jnp.int32),
            pltpu.VMEM((WIN_PER_W, WVOL), jnp.float32),
            pltpu.SemaphoreType.DMA,
        ],
    )
    def body(pk_hbm, out_hbm, pk_v, win_v, sem):
        wid = lax.axis_index("s") * NCORES + lax.axis_index("c")
        b = wid // SUB_PER_B
        i0 = (wid % SUB_PER_B) * WIN_PER_W

        cp = pltpu.async_copy(
            pk_hbm.at[pl.ds(b * ROWS_PER_B, ROWS_PER_B)], pk_v, sem)

        zeros = jnp.zeros((16,), jnp.float32)
        for w in range(WIN_PER_W):
            for k in range(WVOL // 16):
                win_v[w, pl.ds(k * 16, 16)] = zeros
            win_v[w, pl.ds(WVOL - 16, 16)] = zeros

        cp.wait()

        lane = jnp.arange(16, dtype=jnp.int32)

        def gat(q):
            return plsc.load_gather(pk_v, [(q >> 7) + 1, q & 127])

        def acid(col):
            return plsc.bitcast(pk_v[0, pl.ds(col, 16)], jnp.float32)

        def msk(col):
            return plsc.bitcast(pk_v[0, pl.ds(MASK_OFF + col, 16)], jnp.float32)

        chunks = []
        for t in range(3):
            pid3 = (t * 16) * 3 + lane * 3
            cx = 2 * (gat(pid3) + (L - 1))
            cy = 2 * (gat(pid3 + 1) + (L - 1))
            cz = 2 * (gat(pid3 + 2) + (L - 1))
            v = acid(t * 16) * msk(t * 16)
            valid = lane < 16
            chunks.append((cx, cy, cz, v, valid))
        for t in range(3):
            mid = t * 16 + lane
            valid = mid < (L - 1)
            pid3 = jnp.where(valid, mid, L - 2) * 3
            cx = gat(pid3) + gat(pid3 + 3) + 2 * (L - 1)
            cy = gat(pid3 + 1) + gat(pid3 + 4) + 2 * (L - 1)
            cz = gat(pid3 + 2) + gat(pid3 + 5) + 2 * (L - 1)
            v = (acid(t * 16) + acid(t * 16 + 1) + 1.0) * msk(t * 16 + 1)
            chunks.append((cx, cy, cz, v, valid))

        for w in range(WIN_PER_W):
            i3 = (i0 + w) * 3
            sx = jnp.clip(
                2 * (gat(jnp.full((16,), i3, jnp.int32)) + (L - 1))
                - W // 2, 0, D - W)
            sy = jnp.clip(
                2 * (gat(jnp.full((16,), i3 + 1, jnp.int32)) + (L - 1))
                - W // 2, 0, D - W)
            sz = jnp.clip(
                2 * (gat(jnp.full((16,), i3 + 2, jnp.int32)) + (L - 1))
                - W // 2, 0, D - W)
            row = jnp.full((16,), w, jnp.int32)
            for (cx, cy, cz, v, valid) in chunks:
                dx = cx - sx
                dy = cy - sy
                dz = cz - sz
                m = (
                    (dx >= 0) & (dx < W)
                    & (dy >= 0) & (dy < W)
                    & (dz >= 0) & (dz < W)
                    & valid
                )
                off = dx * (W * W) + dy * W + dz
                off = jnp.where(m, off, 0)
                plsc.addupdate_scatter(win_v, [row, off], v, mask=m)

        pltpu.sync_copy(win_v, out_hbm.at[b, pl.ds(i0, WIN_PER_W)])

    return body(packed)


def kernel(acids, mask, idx):
    acids_i = lax.bitcast_convert_type(acids.astype(jnp.float32), jnp.int32)
    mask_i = lax.bitcast_convert_type(mask.astype(jnp.float32), jnp.int32)
    idx_i = idx.astype(jnp.int32).reshape(B, 3 * L)

    z16 = jnp.zeros((B, 16), jnp.int32)
    packed = jnp.concatenate(
        [acids_i, z16, mask_i, z16, idx_i,
         jnp.zeros((B, 512 - 128 - 3 * L), jnp.int32)], axis=1
    ).reshape(ROWS_PER_B * B, 128)

    rows = _sc_windows(packed)
    return rows.reshape(B, L, W, W, W, 1)
